# baseline (speedup 1.0000x reference)
import functools
import math

import jax
import jax.numpy as jnp
from jax.experimental import pallas as pl
from jax.experimental.pallas import tpu as pltpu

_KT = 2048
_NT = 2048
_NB = 2
_TP = 16


def _in_body(n_items, n_rows, x_ref, w_ref, t_ref, fvec_ref, csel_ref,
             ssel_ref, ew_ref, eb_ref, wt_ref, bi_ref, h_ref, ss_ref):
    k = pl.program_id(1)
    nk = pl.num_programs(1)

    @pl.when(k == 0)
    def _init():
        h_ref[...] = jnp.zeros_like(h_ref)
        ss_ref[...] = jnp.zeros_like(ss_ref)

    @pl.when(k < nk - 1)
    def _full_tile():
        xt = x_ref[...]
        h_ref[...] += jnp.dot(xt, w_ref[...],
                              preferred_element_type=jnp.float32)
        ss_ref[...] += jnp.sum(xt * xt, axis=1, keepdims=True)

    @pl.when(k == nk - 1)
    def _last_tile_and_finish():
        xt = x_ref[...]
        col = k * _KT + jax.lax.broadcasted_iota(jnp.int32, xt.shape, 1)
        xt = jnp.where(col < n_items, xt, 0.0)
        wtile = w_ref[...]
        row = k * _KT + jax.lax.broadcasted_iota(jnp.int32, wtile.shape, 0)
        wtile = jnp.where(row < n_rows, wtile, 0.0)
        acc = h_ref[...] + jnp.dot(xt, wtile,
                                   preferred_element_type=jnp.float32)
        ss = ss_ref[...] + jnp.sum(xt * xt, axis=1, keepdims=True)

        t = t_ref[...]
        temp = t * fvec_ref[...]
        te = jnp.cos(temp) * csel_ref[...] + jnp.sin(temp) * ssel_ref[...]
        emb = jnp.dot(te, ew_ref[...],
                      preferred_element_type=jnp.float32) + eb_ref[...]
        contrib = jnp.dot(emb, wt_ref[...],
                          preferred_element_type=jnp.float32)
        rn = jax.lax.rsqrt(jnp.maximum(ss, 1e-24))
        h_ref[...] = jnp.tanh(acc * rn + contrib + bi_ref[...])


def _out_body(h_ref, w_ref, b_ref, o_ref):
    o_ref[...] = jnp.dot(h_ref[...], w_ref[...],
                         preferred_element_type=jnp.float32) + b_ref[...]


def kernel(x, timesteps, emb_W, emb_b, W_in, b_in, W_out, b_out):
    B, N = x.shape
    H = W_in.shape[1]
    T = emb_W.shape[0]
    half = T // 2
    BB = B // _NB

    freqs = jnp.exp(-math.log(10000.0)
                    * jnp.arange(0, half, dtype=jnp.float32) / half)
    fvec = jnp.zeros((1, _TP), jnp.float32)
    fvec = fvec.at[0, :half].set(freqs).at[0, half:T].set(freqs)
    csel = jnp.zeros((1, _TP), jnp.float32).at[0, :half].set(1.0)
    ssel = jnp.zeros((1, _TP), jnp.float32).at[0, half:T].set(1.0)
    ew = jnp.zeros((_TP, _TP), jnp.float32).at[:T, :T].set(emb_W)
    eb = jnp.zeros((1, _TP), jnp.float32).at[0, :T].set(emb_b)
    wt = jnp.zeros((_TP, H), jnp.float32).at[:T, :].set(W_in[N:])
    tf = timesteps.astype(jnp.float32).reshape(B, 1)
    bi = b_in.reshape(1, H)
    bo = b_out.reshape(1, N)

    num_k = pl.cdiv(N, _KT)
    h = pl.pallas_call(
        functools.partial(_in_body, N, N + T),
        grid=(_NB, num_k),
        in_specs=[
            pl.BlockSpec((BB, _KT), lambda b, k: (b, k)),
            pl.BlockSpec((_KT, H), lambda b, k: (k, 0)),
            pl.BlockSpec((BB, 1), lambda b, k: (b, 0)),
            pl.BlockSpec((1, _TP), lambda b, k: (0, 0)),
            pl.BlockSpec((1, _TP), lambda b, k: (0, 0)),
            pl.BlockSpec((1, _TP), lambda b, k: (0, 0)),
            pl.BlockSpec((_TP, _TP), lambda b, k: (0, 0)),
            pl.BlockSpec((1, _TP), lambda b, k: (0, 0)),
            pl.BlockSpec((_TP, H), lambda b, k: (0, 0)),
            pl.BlockSpec((1, H), lambda b, k: (0, 0)),
        ],
        out_specs=pl.BlockSpec((BB, H), lambda b, k: (b, 0)),
        out_shape=jax.ShapeDtypeStruct((B, H), jnp.float32),
        scratch_shapes=[pltpu.VMEM((BB, 1), jnp.float32)],
        compiler_params=pltpu.CompilerParams(
            dimension_semantics=("parallel", "arbitrary")),
    )(x, W_in, tf, fvec, csel, ssel, ew, eb, wt, bi)

    num_j = pl.cdiv(N, _NT)
    out = pl.pallas_call(
        _out_body,
        grid=(num_j,),
        in_specs=[
            pl.BlockSpec((B, H), lambda j: (0, 0)),
            pl.BlockSpec((H, _NT), lambda j: (0, j)),
            pl.BlockSpec((1, _NT), lambda j: (0, j)),
        ],
        out_specs=pl.BlockSpec((B, _NT), lambda j: (0, j)),
        out_shape=jax.ShapeDtypeStruct((B, N), jnp.float32),
        compiler_params=pltpu.CompilerParams(
            dimension_semantics=("parallel",)),
    )(h, W_out, bo)
    return out


# speedup vs baseline: 1.0995x; 1.0995x over previous
import functools
import math

import jax
import jax.numpy as jnp
from jax.experimental import pallas as pl
from jax.experimental.pallas import tpu as pltpu

_KT = 4096
_NT = 4096
_NB = 2
_TP = 16


def _in_body(n_items, n_rows, x_ref, w_ref, t_ref, fvec_ref, csel_ref,
             ssel_ref, ew_ref, eb_ref, wt_ref, bi_ref, h_ref, ss_ref):
    k = pl.program_id(1)
    nk = pl.num_programs(1)

    @pl.when(k == 0)
    def _init():
        h_ref[...] = jnp.zeros_like(h_ref)
        ss_ref[...] = jnp.zeros_like(ss_ref)

    @pl.when(k < nk - 1)
    def _full_tile():
        xt = x_ref[...]
        h_ref[...] += jnp.dot(xt, w_ref[...],
                              preferred_element_type=jnp.float32)
        xf = xt.astype(jnp.float32)
        ss_ref[...] += jnp.sum(xf * xf, axis=1, keepdims=True)

    @pl.when(k == nk - 1)
    def _last_tile_and_finish():
        xt = x_ref[...]
        col = k * _KT + jax.lax.broadcasted_iota(jnp.int32, xt.shape, 1)
        xt = jnp.where(col < n_items, xt, jnp.bfloat16(0))
        wtile = w_ref[...]
        row = k * _KT + jax.lax.broadcasted_iota(jnp.int32, wtile.shape, 0)
        wtile = jnp.where(row < n_rows, wtile, jnp.bfloat16(0))
        acc = h_ref[...] + jnp.dot(xt, wtile,
                                   preferred_element_type=jnp.float32)
        xf = xt.astype(jnp.float32)
        ss = ss_ref[...] + jnp.sum(xf * xf, axis=1, keepdims=True)

        t = t_ref[...]
        temp = t * fvec_ref[...]
        te = jnp.cos(temp) * csel_ref[...] + jnp.sin(temp) * ssel_ref[...]
        emb = jnp.dot(te, ew_ref[...],
                      preferred_element_type=jnp.float32) + eb_ref[...]
        contrib = jnp.dot(emb, wt_ref[...],
                          preferred_element_type=jnp.float32)
        rn = jax.lax.rsqrt(jnp.maximum(ss, 1e-24))
        h_ref[...] = jnp.tanh(acc * rn + contrib + bi_ref[...])


def _out_body(h_ref, w_ref, b_ref, o_ref):
    acc = jnp.dot(h_ref[...].astype(jnp.bfloat16), w_ref[...],
                  preferred_element_type=jnp.float32)
    o_ref[...] = (acc + b_ref[...]).astype(jnp.bfloat16)


def kernel(x, timesteps, emb_W, emb_b, W_in, b_in, W_out, b_out):
    B, N = x.shape
    H = W_in.shape[1]
    T = emb_W.shape[0]
    half = T // 2
    BB = B // _NB

    xb = x.astype(jnp.bfloat16)
    Wib = W_in.astype(jnp.bfloat16)
    Wob = W_out.astype(jnp.bfloat16)
    freqs = jnp.exp(-math.log(10000.0)
                    * jnp.arange(0, half, dtype=jnp.float32) / half)
    fvec = jnp.zeros((1, _TP), jnp.float32)
    fvec = fvec.at[0, :half].set(freqs).at[0, half:T].set(freqs)
    csel = jnp.zeros((1, _TP), jnp.float32).at[0, :half].set(1.0)
    ssel = jnp.zeros((1, _TP), jnp.float32).at[0, half:T].set(1.0)
    ew = jnp.zeros((_TP, _TP), jnp.float32).at[:T, :T].set(emb_W)
    eb = jnp.zeros((1, _TP), jnp.float32).at[0, :T].set(emb_b)
    wt = jnp.zeros((_TP, H), jnp.float32).at[:T, :].set(W_in[N:])
    tf = timesteps.astype(jnp.float32).reshape(B, 1)
    bi = b_in.reshape(1, H)
    bo = b_out.reshape(1, N)

    num_k = pl.cdiv(N, _KT)
    h = pl.pallas_call(
        functools.partial(_in_body, N, N + T),
        grid=(_NB, num_k),
        in_specs=[
            pl.BlockSpec((BB, _KT), lambda b, k: (b, k)),
            pl.BlockSpec((_KT, H), lambda b, k: (k, 0)),
            pl.BlockSpec((BB, 1), lambda b, k: (b, 0)),
            pl.BlockSpec((1, _TP), lambda b, k: (0, 0)),
            pl.BlockSpec((1, _TP), lambda b, k: (0, 0)),
            pl.BlockSpec((1, _TP), lambda b, k: (0, 0)),
            pl.BlockSpec((_TP, _TP), lambda b, k: (0, 0)),
            pl.BlockSpec((1, _TP), lambda b, k: (0, 0)),
            pl.BlockSpec((_TP, H), lambda b, k: (0, 0)),
            pl.BlockSpec((1, H), lambda b, k: (0, 0)),
        ],
        out_specs=pl.BlockSpec((BB, H), lambda b, k: (b, 0)),
        out_shape=jax.ShapeDtypeStruct((B, H), jnp.float32),
        scratch_shapes=[pltpu.VMEM((BB, 1), jnp.float32)],
        compiler_params=pltpu.CompilerParams(
            dimension_semantics=("parallel", "arbitrary")),
    )(xb, Wib, tf, fvec, csel, ssel, ew, eb, wt, bi)

    num_j = pl.cdiv(N, _NT)
    outb = pl.pallas_call(
        _out_body,
        grid=(num_j,),
        in_specs=[
            pl.BlockSpec((B, H), lambda j: (0, 0)),
            pl.BlockSpec((H, _NT), lambda j: (0, j)),
            pl.BlockSpec((1, _NT), lambda j: (0, j)),
        ],
        out_specs=pl.BlockSpec((B, _NT), lambda j: (0, j)),
        out_shape=jax.ShapeDtypeStruct((B, N), jnp.bfloat16),
        compiler_params=pltpu.CompilerParams(
            dimension_semantics=("parallel",)),
    )(h, Wob, bo)
    return outb.astype(jnp.float32)


# speedup vs baseline: 2.9440x; 2.6776x over previous
import functools
import math

import jax
import jax.numpy as jnp
from jax.experimental import pallas as pl
from jax.experimental.pallas import tpu as pltpu

_KT = 2048
_NT = 2048
_TP = 16


def _in_body(n_items, n_rows, x_ref, w_ref, t_ref, fvec_ref, csel_ref,
             ssel_ref, ew_ref, eb_ref, wt_ref, bi_ref, h_ref, ss_ref):
    k = pl.program_id(0)
    nk = pl.num_programs(0)

    @pl.when(k == 0)
    def _init():
        h_ref[...] = jnp.zeros_like(h_ref)
        ss_ref[...] = jnp.zeros_like(ss_ref)

    @pl.when(k < nk - 1)
    def _full_tile():
        xt = x_ref[...]
        h_ref[...] += jnp.dot(w_ref[...], xt,
                              preferred_element_type=jnp.float32)
        ss_ref[...] += jnp.sum(xt * xt, axis=0, keepdims=True)

    @pl.when(k == nk - 1)
    def _last_tile_and_finish():
        xt = x_ref[...]
        row = k * _KT + jax.lax.broadcasted_iota(jnp.int32, xt.shape, 0)
        xt = jnp.where(row < n_items, xt, 0.0)
        wtile = w_ref[...]
        col = k * _KT + jax.lax.broadcasted_iota(jnp.int32, wtile.shape, 1)
        wtile = jnp.where(col < n_rows, wtile, 0.0)
        acc = h_ref[...] + jnp.dot(wtile, xt,
                                   preferred_element_type=jnp.float32)
        ss = ss_ref[...] + jnp.sum(xt * xt, axis=0, keepdims=True)

        t = t_ref[...]
        temp = fvec_ref[...] * t
        te = jnp.cos(temp) * csel_ref[...] + jnp.sin(temp) * ssel_ref[...]
        emb = jnp.dot(ew_ref[...], te,
                      preferred_element_type=jnp.float32) + eb_ref[...]
        contrib = jnp.dot(wt_ref[...], emb,
                          preferred_element_type=jnp.float32)
        rn = jax.lax.rsqrt(jnp.maximum(ss, 1e-24))
        h_ref[...] = jnp.tanh(acc * rn + contrib + bi_ref[...])


def _out_body(h_ref, w_ref, b_ref, o_ref):
    o_ref[...] = jnp.dot(w_ref[...], h_ref[...],
                         preferred_element_type=jnp.float32) + b_ref[...]


def kernel(x, timesteps, emb_W, emb_b, W_in, b_in, W_out, b_out):
    B, N = x.shape
    H = W_in.shape[1]
    T = emb_W.shape[0]
    half = T // 2

    xT = x.T
    WiT = W_in.T
    WoT = W_out.T
    boT = b_out.reshape(N, 1)
    freqs = jnp.exp(-math.log(10000.0)
                    * jnp.arange(0, half, dtype=jnp.float32) / half)
    fvec = jnp.zeros((_TP, 1), jnp.float32)
    fvec = fvec.at[:half, 0].set(freqs).at[half:T, 0].set(freqs)
    csel = jnp.zeros((_TP, 1), jnp.float32).at[:half, 0].set(1.0)
    ssel = jnp.zeros((_TP, 1), jnp.float32).at[half:T, 0].set(1.0)
    ew = jnp.zeros((_TP, _TP), jnp.float32).at[:T, :T].set(emb_W.T)
    eb = jnp.zeros((_TP, 1), jnp.float32).at[:T, 0].set(emb_b)
    wt = jnp.zeros((H, _TP), jnp.float32).at[:, :T].set(W_in[N:].T)
    tf = timesteps.astype(jnp.float32).reshape(1, B)
    bi = b_in.reshape(H, 1)

    num_k = pl.cdiv(N, _KT)
    hT = pl.pallas_call(
        functools.partial(_in_body, N, N + T),
        grid=(num_k,),
        in_specs=[
            pl.BlockSpec((_KT, B), lambda k: (k, 0)),
            pl.BlockSpec((H, _KT), lambda k: (0, k)),
            pl.BlockSpec((1, B), lambda k: (0, 0)),
            pl.BlockSpec((_TP, 1), lambda k: (0, 0)),
            pl.BlockSpec((_TP, 1), lambda k: (0, 0)),
            pl.BlockSpec((_TP, 1), lambda k: (0, 0)),
            pl.BlockSpec((_TP, _TP), lambda k: (0, 0)),
            pl.BlockSpec((_TP, 1), lambda k: (0, 0)),
            pl.BlockSpec((H, _TP), lambda k: (0, 0)),
            pl.BlockSpec((H, 1), lambda k: (0, 0)),
        ],
        out_specs=pl.BlockSpec((H, B), lambda k: (0, 0)),
        out_shape=jax.ShapeDtypeStruct((H, B), jnp.float32),
        scratch_shapes=[pltpu.VMEM((1, B), jnp.float32)],
        compiler_params=pltpu.CompilerParams(
            dimension_semantics=("arbitrary",)),
    )(xT, WiT, tf, fvec, csel, ssel, ew, eb, wt, bi)

    num_j = pl.cdiv(N, _NT)
    outT = pl.pallas_call(
        _out_body,
        grid=(num_j,),
        in_specs=[
            pl.BlockSpec((H, B), lambda j: (0, 0)),
            pl.BlockSpec((_NT, H), lambda j: (j, 0)),
            pl.BlockSpec((_NT, 1), lambda j: (j, 0)),
        ],
        out_specs=pl.BlockSpec((_NT, B), lambda j: (j, 0)),
        out_shape=jax.ShapeDtypeStruct((N, B), jnp.float32),
        compiler_params=pltpu.CompilerParams(
            dimension_semantics=("parallel",)),
    )(hT, WoT, boT)
    return outT.T


# speedup vs baseline: 3.4800x; 1.1821x over previous
import functools
import math

import jax
import jax.numpy as jnp
from jax.experimental import pallas as pl
from jax.experimental.pallas import tpu as pltpu

_KT = 2048
_NT = 4096
_TP = 16


def _in_body(n_items, n_rows, x_ref, w_ref, t_ref, fvec_ref, csel_ref,
             ssel_ref, ew_ref, eb_ref, wt_ref, bi_ref, h_ref, ss_ref):
    k = pl.program_id(0)
    nk = pl.num_programs(0)

    @pl.when(k == 0)
    def _init():
        h_ref[...] = jnp.zeros_like(h_ref)
        ss_ref[...] = jnp.zeros_like(ss_ref)

    @pl.when(k < nk - 1)
    def _full_tile():
        xt = x_ref[...]
        h_ref[...] += jnp.dot(w_ref[...], xt,
                              preferred_element_type=jnp.float32)
        ss_ref[...] += jnp.sum(xt * xt, axis=0, keepdims=True)

    @pl.when(k == nk - 1)
    def _last_tile_and_finish():
        xt = x_ref[...]
        row = k * _KT + jax.lax.broadcasted_iota(jnp.int32, xt.shape, 0)
        xt = jnp.where(row < n_items, xt, 0.0)
        wtile = w_ref[...]
        col = k * _KT + jax.lax.broadcasted_iota(jnp.int32, wtile.shape, 1)
        wtile = jnp.where(col < n_rows, wtile, 0.0)
        acc = h_ref[...] + jnp.dot(wtile, xt,
                                   preferred_element_type=jnp.float32)
        ss = ss_ref[...] + jnp.sum(xt * xt, axis=0, keepdims=True)

        t = t_ref[...]
        temp = fvec_ref[...] * t
        te = jnp.cos(temp) * csel_ref[...] + jnp.sin(temp) * ssel_ref[...]
        emb = jnp.dot(ew_ref[...], te,
                      preferred_element_type=jnp.float32) + eb_ref[...]
        contrib = jnp.dot(wt_ref[...], emb,
                          preferred_element_type=jnp.float32)
        rn = jax.lax.rsqrt(jnp.maximum(ss, 1e-24))
        h_ref[...] = jnp.tanh(acc * rn + contrib + bi_ref[...])


def _out_body(h_ref, w_ref, b_ref, o_ref):
    bcol = b_ref[...].T
    o_ref[...] = jnp.dot(w_ref[...], h_ref[...],
                         preferred_element_type=jnp.float32) + bcol


def kernel(x, timesteps, emb_W, emb_b, W_in, b_in, W_out, b_out):
    B, N = x.shape
    H = W_in.shape[1]
    T = emb_W.shape[0]
    half = T // 2

    xT = x.T
    WiT = W_in.T
    WoT = W_out.T
    boT = b_out.reshape(1, N)
    freqs = jnp.exp(-math.log(10000.0)
                    * jnp.arange(0, half, dtype=jnp.float32) / half)
    fvec = jnp.zeros((_TP, 1), jnp.float32)
    fvec = fvec.at[:half, 0].set(freqs).at[half:T, 0].set(freqs)
    csel = jnp.zeros((_TP, 1), jnp.float32).at[:half, 0].set(1.0)
    ssel = jnp.zeros((_TP, 1), jnp.float32).at[half:T, 0].set(1.0)
    ew = jnp.zeros((_TP, _TP), jnp.float32).at[:T, :T].set(emb_W.T)
    eb = jnp.zeros((_TP, 1), jnp.float32).at[:T, 0].set(emb_b)
    wt = jnp.zeros((H, _TP), jnp.float32).at[:, :T].set(W_in[N:].T)
    tf = timesteps.astype(jnp.float32).reshape(1, B)
    bi = b_in.reshape(H, 1)

    num_k = pl.cdiv(N, _KT)
    hT = pl.pallas_call(
        functools.partial(_in_body, N, N + T),
        grid=(num_k,),
        in_specs=[
            pl.BlockSpec((_KT, B), lambda k: (k, 0)),
            pl.BlockSpec((H, _KT), lambda k: (0, k)),
            pl.BlockSpec((1, B), lambda k: (0, 0)),
            pl.BlockSpec((_TP, 1), lambda k: (0, 0)),
            pl.BlockSpec((_TP, 1), lambda k: (0, 0)),
            pl.BlockSpec((_TP, 1), lambda k: (0, 0)),
            pl.BlockSpec((_TP, _TP), lambda k: (0, 0)),
            pl.BlockSpec((_TP, 1), lambda k: (0, 0)),
            pl.BlockSpec((H, _TP), lambda k: (0, 0)),
            pl.BlockSpec((H, 1), lambda k: (0, 0)),
        ],
        out_specs=pl.BlockSpec((H, B), lambda k: (0, 0)),
        out_shape=jax.ShapeDtypeStruct((H, B), jnp.float32),
        scratch_shapes=[pltpu.VMEM((1, B), jnp.float32)],
        compiler_params=pltpu.CompilerParams(
            dimension_semantics=("arbitrary",)),
    )(xT, WiT, tf, fvec, csel, ssel, ew, eb, wt, bi)

    num_j = pl.cdiv(N, _NT)
    outT = pl.pallas_call(
        _out_body,
        grid=(num_j,),
        in_specs=[
            pl.BlockSpec((H, B), lambda j: (0, 0)),
            pl.BlockSpec((_NT, H), lambda j: (j, 0)),
            pl.BlockSpec((1, _NT), lambda j: (0, j)),
        ],
        out_specs=pl.BlockSpec((_NT, B), lambda j: (j, 0)),
        out_shape=jax.ShapeDtypeStruct((N, B), jnp.float32),
        compiler_params=pltpu.CompilerParams(
            dimension_semantics=("parallel",)),
    )(hT, WoT, boT)
    return outT.T
